# baseline (speedup 1.0000x reference)
import functools

import jax
import jax.numpy as jnp
from jax import lax
from jax.experimental import pallas as pl
from jax.experimental.pallas import tpu as pltpu
from jax.experimental.pallas import tpu_sc as plsc

N = 10000
E = 320000
D = 128
ED = 16
H = 128

NC = 2
NS = 16
NW = NC * NS
K = 5
ES = E // K
EWS = ES // NW
C = 80
NCH = EWS // C
N_PAD = 10240
NROWS = N_PAD // NS


@functools.cache
def _build_sc_kernels():
    mesh = plsc.VectorSubcoreMesh(
        core_axis_name="c", subcore_axis_name="s",
        num_cores=NC, num_subcores=NS)

    @functools.partial(
        pl.kernel,
        out_type=(jax.ShapeDtypeStruct((ES, D), jnp.float32),
                  jax.ShapeDtypeStruct((ES, D), jnp.float32)),
        mesh=mesh,
        scratch_types=[
            pltpu.VMEM((NCH, C), jnp.int32),
            pltpu.VMEM((NCH, C), jnp.int32),
            pltpu.VMEM((2, C, D), jnp.float32),
            pltpu.VMEM((2, C, D), jnp.float32),
            pltpu.SemaphoreType.DMA,
            pltpu.SemaphoreType.DMA,
            pltpu.SemaphoreType.DMA,
            pltpu.SemaphoreType.DMA,
        ],
    )
    def sc_gather(row_hbm, col_hbm, x_hbm, xr_hbm, xc_hbm,
                  ridx_v, cidx_v, buf_r, buf_c, gsem0, gsem1, ssem0, ssem1):
        wid = lax.axis_index("s") * NC + lax.axis_index("c")
        base = wid * EWS
        pltpu.sync_copy(row_hbm.at[wid], ridx_v)
        pltpu.sync_copy(col_hbm.at[wid], cidx_v)

        gsem = (gsem0, gsem1)
        ssem = (ssem0, ssem1)

        def start_gather(ch, b):
            pltpu.async_copy(x_hbm.at[ridx_v.at[ch]], buf_r.at[b], gsem[b])
            pltpu.async_copy(x_hbm.at[cidx_v.at[ch]], buf_c.at[b], gsem[b])

        def wait_gather(b):
            pltpu.make_async_copy(x_hbm.at[ridx_v.at[0]], buf_r.at[b],
                                  gsem[b]).wait()
            pltpu.make_async_copy(x_hbm.at[cidx_v.at[0]], buf_c.at[b],
                                  gsem[b]).wait()

        def start_store(ch, b):
            pltpu.async_copy(buf_r.at[b], xr_hbm.at[pl.ds(base + ch * C, C)],
                             ssem[b])
            pltpu.async_copy(buf_c.at[b], xc_hbm.at[pl.ds(base + ch * C, C)],
                             ssem[b])

        def wait_store(b):
            pltpu.make_async_copy(buf_r.at[b], xr_hbm.at[pl.ds(base, C)],
                                  ssem[b]).wait()
            pltpu.make_async_copy(buf_c.at[b], xc_hbm.at[pl.ds(base, C)],
                                  ssem[b]).wait()

        start_gather(0, 0)

        def body(i, _):
            ch = 2 * i
            start_gather(ch + 1, 1)
            wait_gather(0)
            start_store(ch, 0)
            wait_store(0)
            start_gather(ch + 2, 0)
            wait_gather(1)
            start_store(ch + 1, 1)
            wait_store(1)
            return 0

        lax.fori_loop(0, NCH // 2, body, 0)
        wait_gather(0)
        start_store(NCH - 1, 0)
        wait_store(0)

    @functools.partial(
        pl.kernel,
        out_type=jax.ShapeDtypeStruct((NC, N_PAD, D), jnp.float32),
        mesh=mesh,
        scratch_types=[
            pltpu.VMEM((K, NCH, C), jnp.int32),
            pltpu.VMEM((2, C, D), jnp.float32),
            pltpu.VMEM_SHARED((N_PAD, D), jnp.float32),
            pltpu.SemaphoreType.DMA,
            pltpu.SemaphoreType.DMA,
            pltpu.SemaphoreType.DMA,
            pltpu.SemaphoreType.DMA,
        ],
    )
    def sc_scatter(col_hbm, eo0, eo1, eo2, eo3, eo4, zeros_hbm, out_hbm,
                   cidx_v, buf, aggr_sh, lsem0, lsem1, asem0, asem1):
        c = lax.axis_index("c")
        s = lax.axis_index("s")
        wid = s * NC + c
        base = wid * EWS
        rbase = s * NROWS

        pltpu.sync_copy(zeros_hbm, buf.at[0])

        def zbody(k, _):
            pltpu.sync_copy(buf.at[0], aggr_sh.at[pl.ds(rbase + k * C, C)])
            return 0

        lax.fori_loop(0, NROWS // C, zbody, 0)
        plsc.subcore_barrier()

        pltpu.sync_copy(col_hbm.at[wid], cidx_v)

        lsem = (lsem0, lsem1)
        asem = (asem0, asem1)

        def start_load(eo, ch, b):
            pltpu.async_copy(eo.at[pl.ds(base + ch * C, C)], buf.at[b],
                             lsem[b])

        def wait_load(eo, b):
            pltpu.make_async_copy(eo.at[pl.ds(base, C)], buf.at[b],
                                  lsem[b]).wait()

        def do_add(sl, ch, b):
            pltpu.async_copy(buf.at[b], aggr_sh.at[cidx_v.at[sl].at[ch]],
                             asem[b], add=True)
            pltpu.make_async_copy(buf.at[b], aggr_sh.at[pl.ds(0, C)],
                                  asem[b]).wait()

        for sl, eo in enumerate((eo0, eo1, eo2, eo3, eo4)):
            start_load(eo, 0, 0)

            def body(i, _, eo=eo, sl=sl):
                ch = 2 * i
                start_load(eo, ch + 1, 1)
                wait_load(eo, 0)
                do_add(sl, ch, 0)
                start_load(eo, ch + 2, 0)
                wait_load(eo, 1)
                do_add(sl, ch + 1, 1)
                return 0

            lax.fori_loop(0, NCH // 2, body, 0)
            wait_load(eo, 0)
            do_add(sl, NCH - 1, 0)

        plsc.subcore_barrier()

        def wbody(k, _):
            pltpu.sync_copy(aggr_sh.at[pl.ds(rbase + k * C, C)], buf.at[0])
            pltpu.sync_copy(buf.at[0],
                            out_hbm.at[c].at[pl.ds(rbase + k * C, C)])
            return 0

        lax.fori_loop(0, NROWS // C, wbody, 0)

    return sc_gather, sc_scatter


BE = 2000


def _edge_mlp_body(xr, xc, ea, w1r, w1c, w1e, b1, w2, b2, w3, b3, out):
    bf = jnp.bfloat16
    h = (jnp.dot(xr[...].astype(bf), w1r[...],
                 preferred_element_type=jnp.float32)
         + jnp.dot(xc[...].astype(bf), w1c[...],
                   preferred_element_type=jnp.float32)
         + jnp.dot(ea[...].astype(bf), w1e[...],
                   preferred_element_type=jnp.float32)
         + b1[...])
    h = jnp.maximum(h, 0.0).astype(bf)
    h = jnp.maximum(
        jnp.dot(h, w2[...], preferred_element_type=jnp.float32) + b2[...],
        0.0).astype(bf)
    out[...] = jnp.dot(h, w3[...], preferred_element_type=jnp.float32) + b3[...]


def _edge_mlp(xr, xc, ea, w1r, w1c, w1e, b1, w2, b2, w3, b3):
    full = lambda shape: pl.BlockSpec(shape, lambda i: (0, 0))
    return pl.pallas_call(
        _edge_mlp_body,
        grid=(ES // BE,),
        in_specs=[
            pl.BlockSpec((BE, D), lambda i: (i, 0)),
            pl.BlockSpec((BE, D), lambda i: (i, 0)),
            pl.BlockSpec((BE, ED), lambda i: (i, 0)),
            full((D, H)), full((D, H)), full((ED, H)), full((1, H)),
            full((H, H)), full((1, H)),
            full((H, D)), full((1, D)),
        ],
        out_specs=pl.BlockSpec((BE, D), lambda i: (i, 0)),
        out_shape=jax.ShapeDtypeStruct((ES, D), jnp.float32),
    )(xr, xc, ea, w1r, w1c, w1e, b1, w2, b2, w3, b3)


BN = 1000


def _node_body(x, p0, p1, wn1a, wn1b, bn1, wn2, bn2, gamma, beta, out):
    xb = x[...]
    aggr = p0[...] + p1[...]
    g = jnp.maximum(
        jnp.dot(xb, wn1a[...], preferred_element_type=jnp.float32)
        + jnp.dot(aggr, wn1b[...], preferred_element_type=jnp.float32)
        + bn1[...], 0.0)
    h = xb + jnp.dot(g, wn2[...], preferred_element_type=jnp.float32) + bn2[...]
    mu = jnp.mean(h, axis=-1, keepdims=True)
    var = jnp.mean((h - mu) ** 2, axis=-1, keepdims=True)
    out[...] = (h - mu) * jax.lax.rsqrt(var + 1e-5) * gamma[...] + beta[...]


def _node_mlp(x, p0, p1, wn1a, wn1b, bn1, wn2, bn2, gamma, beta):
    full = lambda shape: pl.BlockSpec(shape, lambda i: (0, 0))
    return pl.pallas_call(
        _node_body,
        grid=(N // BN,),
        in_specs=[
            pl.BlockSpec((BN, D), lambda i: (i, 0)),
            pl.BlockSpec((BN, D), lambda i: (i, 0)),
            pl.BlockSpec((BN, D), lambda i: (i, 0)),
            full((D, H)), full((D, H)), full((1, H)),
            full((H, D)), full((1, D)),
            full((1, D)), full((1, D)),
        ],
        out_specs=pl.BlockSpec((BN, D), lambda i: (i, 0)),
        out_shape=jax.ShapeDtypeStruct((N, D), jnp.float32),
    )(x, p0, p1, wn1a, wn1b, bn1, wn2, bn2, gamma, beta)


def kernel(x, edge_index, edge_attr, W1, b1, W2, b2, W3, b3,
           Wn1, bn1, Wn2, bn2, gamma, beta):
    row = edge_index[0].astype(jnp.int32).reshape(K, NW, NCH, C)
    col = edge_index[1].astype(jnp.int32).reshape(K, NW, NCH, C)
    col_sc = col.transpose(1, 0, 2, 3)
    ea3 = edge_attr.reshape(K, ES, ED)

    sc_gather, sc_scatter = _build_sc_kernels()

    bf = jnp.bfloat16
    w1r = W1[:D].astype(bf)
    w1c = W1[D:2 * D].astype(bf)
    w1e = W1[2 * D:].astype(bf)
    b1r = b1.reshape(1, H)
    w2 = W2.astype(bf)
    b2r = b2.reshape(1, H)
    w3 = W3.astype(bf)
    b3r = b3.reshape(1, D)

    eos = []
    for sl in range(K):
        xr, xc = sc_gather(row[sl], col[sl], x)
        eos.append(_edge_mlp(
            xr, xc, ea3[sl],
            w1r, w1c, w1e, b1r, w2, b2r, w3, b3r))

    partials = sc_scatter(col_sc, *eos, jnp.zeros((C, D), jnp.float32))

    return _node_mlp(
        x, partials[0], partials[1],
        Wn1[:D], Wn1[D:], bn1.reshape(1, H),
        Wn2, bn2.reshape(1, D),
        gamma.reshape(1, D), beta.reshape(1, D))


# speedup vs baseline: 1.0463x; 1.0463x over previous
import functools

import jax
import jax.numpy as jnp
from jax import lax
from jax.experimental import pallas as pl
from jax.experimental.pallas import tpu as pltpu
from jax.experimental.pallas import tpu_sc as plsc

N = 10000
E = 320000
D = 128
ED = 16
H = 128

NC = 2
NS = 16
NW = NC * NS
EW = E // NW
C = 80
NCHUNK = EW // C
N_PAD = 10240
NROWS = N_PAD // NS
NB = 4
NB_S = 3


@functools.cache
def _build_sc_kernels():
    mesh = plsc.VectorSubcoreMesh(
        core_axis_name="c", subcore_axis_name="s",
        num_cores=NC, num_subcores=NS)

    @functools.partial(
        pl.kernel,
        out_type=(jax.ShapeDtypeStruct((E, D), jnp.float32),
                  jax.ShapeDtypeStruct((E, D), jnp.float32)),
        mesh=mesh,
        scratch_types=[
            pltpu.VMEM((NCHUNK, C), jnp.int32),
            pltpu.VMEM((NCHUNK, C), jnp.int32),
            pltpu.VMEM((NB, C, D), jnp.float32),
            pltpu.VMEM((NB, C, D), jnp.float32),
        ] + [pltpu.SemaphoreType.DMA] * (2 * NB),
    )
    def sc_gather(row_hbm, col_hbm, x_hbm, xr_hbm, xc_hbm,
                  ridx_v, cidx_v, buf_r, buf_c, *sems):
        wid = lax.axis_index("s") * NC + lax.axis_index("c")
        base = wid * EW
        pltpu.sync_copy(row_hbm.at[wid], ridx_v)
        pltpu.sync_copy(col_hbm.at[wid], cidx_v)

        gsem = sems[:NB]
        ssem = sems[NB:]

        def start_gather(ch, b):
            pltpu.async_copy(x_hbm.at[ridx_v.at[ch]], buf_r.at[b], gsem[b])
            pltpu.async_copy(x_hbm.at[cidx_v.at[ch]], buf_c.at[b], gsem[b])

        def wait_gather(b):
            pltpu.make_async_copy(x_hbm.at[ridx_v.at[0]], buf_r.at[b],
                                  gsem[b]).wait()
            pltpu.make_async_copy(x_hbm.at[cidx_v.at[0]], buf_c.at[b],
                                  gsem[b]).wait()

        def start_store(ch, b):
            pltpu.async_copy(buf_r.at[b], xr_hbm.at[pl.ds(base + ch * C, C)],
                             ssem[b])
            pltpu.async_copy(buf_c.at[b], xc_hbm.at[pl.ds(base + ch * C, C)],
                             ssem[b])

        def wait_store(b):
            pltpu.make_async_copy(buf_r.at[b], xr_hbm.at[pl.ds(base, C)],
                                  ssem[b]).wait()
            pltpu.make_async_copy(buf_c.at[b], xc_hbm.at[pl.ds(base, C)],
                                  ssem[b]).wait()

        for b in range(NB):
            start_gather(b, b)

        def body(i, _):
            ch = NB * i
            for k in range(NB):
                wait_gather(k)
                start_store(ch + k, k)
            for k in range(NB):
                wait_store(k)

                @pl.when(ch + NB + k < NCHUNK)
                def _(k=k):
                    start_gather(ch + NB + k, k)

            return 0

        lax.fori_loop(0, NCHUNK // NB, body, 0)
        wait_gather(0)
        start_store(NCHUNK - 1, 0)
        wait_store(0)

    @functools.partial(
        pl.kernel,
        out_type=jax.ShapeDtypeStruct((NC, N_PAD, D), jnp.float32),
        mesh=mesh,
        scratch_types=[
            pltpu.VMEM((NCHUNK, C), jnp.int32),
            pltpu.VMEM((NB_S, C, D), jnp.float32),
            pltpu.VMEM_SHARED((N_PAD, D), jnp.float32),
        ] + [pltpu.SemaphoreType.DMA] * (2 * NB_S),
    )
    def sc_scatter(col_hbm, eo_hbm, zeros_hbm, out_hbm,
                   cidx_v, buf, aggr_sh, *sems):
        c = lax.axis_index("c")
        s = lax.axis_index("s")
        wid = s * NC + c
        base = wid * EW
        rbase = s * NROWS

        pltpu.sync_copy(zeros_hbm, buf.at[0])

        def zbody(k, _):
            pltpu.sync_copy(buf.at[0], aggr_sh.at[pl.ds(rbase + k * C, C)])
            return 0

        lax.fori_loop(0, NROWS // C, zbody, 0)
        plsc.subcore_barrier()

        pltpu.sync_copy(col_hbm.at[wid], cidx_v)

        lsem = sems[:NB_S]
        asem = sems[NB_S:]

        def start_load(ch, b):
            pltpu.async_copy(eo_hbm.at[pl.ds(base + ch * C, C)], buf.at[b],
                             lsem[b])

        def wait_load(b):
            pltpu.make_async_copy(eo_hbm.at[pl.ds(base, C)], buf.at[b],
                                  lsem[b]).wait()

        def do_add(ch, b):
            pltpu.async_copy(buf.at[b], aggr_sh.at[cidx_v.at[ch]], asem[b],
                             add=True)
            pltpu.make_async_copy(buf.at[b], aggr_sh.at[pl.ds(0, C)],
                                  asem[b]).wait()

        for b in range(NB_S):
            start_load(b, b)

        def body(i, _):
            ch = NB_S * i
            for k in range(NB_S):
                wait_load(k)
                do_add(ch + k, k)

                @pl.when(ch + NB_S + k < NCHUNK)
                def _(k=k):
                    start_load(ch + NB_S + k, k)

            return 0

        lax.fori_loop(0, NCHUNK // NB_S, body, 0)
        wait_load(0)
        do_add(NCHUNK - 2, 0)
        wait_load(1)
        do_add(NCHUNK - 1, 1)
        plsc.subcore_barrier()

        def wbody(k, _):
            pltpu.sync_copy(aggr_sh.at[pl.ds(rbase + k * C, C)], buf.at[0])
            pltpu.sync_copy(buf.at[0],
                            out_hbm.at[c].at[pl.ds(rbase + k * C, C)])
            return 0

        lax.fori_loop(0, NROWS // C, wbody, 0)

    return sc_gather, sc_scatter


BE = 2000


def _edge_mlp_body(xr, xc, ea, w1r, w1c, w1e, b1, w2, b2, w3, b3, out):
    bf = jnp.bfloat16
    h = (jnp.dot(xr[...].astype(bf), w1r[...],
                 preferred_element_type=jnp.float32)
         + jnp.dot(xc[...].astype(bf), w1c[...],
                   preferred_element_type=jnp.float32)
         + jnp.dot(ea[...].astype(bf), w1e[...],
                   preferred_element_type=jnp.float32)
         + b1[...])
    h = jnp.maximum(h, 0.0).astype(bf)
    h = jnp.maximum(
        jnp.dot(h, w2[...], preferred_element_type=jnp.float32) + b2[...],
        0.0).astype(bf)
    out[...] = jnp.dot(h, w3[...], preferred_element_type=jnp.float32) + b3[...]


def _edge_mlp(xr, xc, ea, w1r, w1c, w1e, b1, w2, b2, w3, b3):
    full = lambda shape: pl.BlockSpec(shape, lambda i: (0, 0))
    return pl.pallas_call(
        _edge_mlp_body,
        grid=(E // BE,),
        in_specs=[
            pl.BlockSpec((BE, D), lambda i: (i, 0)),
            pl.BlockSpec((BE, D), lambda i: (i, 0)),
            pl.BlockSpec((BE, ED), lambda i: (i, 0)),
            full((D, H)), full((D, H)), full((ED, H)), full((1, H)),
            full((H, H)), full((1, H)),
            full((H, D)), full((1, D)),
        ],
        out_specs=pl.BlockSpec((BE, D), lambda i: (i, 0)),
        out_shape=jax.ShapeDtypeStruct((E, D), jnp.float32),
    )(xr, xc, ea, w1r, w1c, w1e, b1, w2, b2, w3, b3)


BN = 1000


def _node_body(x, p0, p1, wn1a, wn1b, bn1, wn2, bn2, gamma, beta, out):
    xb = x[...]
    aggr = p0[...] + p1[...]
    g = jnp.maximum(
        jnp.dot(xb, wn1a[...], preferred_element_type=jnp.float32)
        + jnp.dot(aggr, wn1b[...], preferred_element_type=jnp.float32)
        + bn1[...], 0.0)
    h = xb + jnp.dot(g, wn2[...], preferred_element_type=jnp.float32) + bn2[...]
    mu = jnp.mean(h, axis=-1, keepdims=True)
    var = jnp.mean((h - mu) ** 2, axis=-1, keepdims=True)
    out[...] = (h - mu) * jax.lax.rsqrt(var + 1e-5) * gamma[...] + beta[...]


def _node_mlp(x, p0, p1, wn1a, wn1b, bn1, wn2, bn2, gamma, beta):
    full = lambda shape: pl.BlockSpec(shape, lambda i: (0, 0))
    return pl.pallas_call(
        _node_body,
        grid=(N // BN,),
        in_specs=[
            pl.BlockSpec((BN, D), lambda i: (i, 0)),
            pl.BlockSpec((BN, D), lambda i: (i, 0)),
            pl.BlockSpec((BN, D), lambda i: (i, 0)),
            full((D, H)), full((D, H)), full((1, H)),
            full((H, D)), full((1, D)),
            full((1, D)), full((1, D)),
        ],
        out_specs=pl.BlockSpec((BN, D), lambda i: (i, 0)),
        out_shape=jax.ShapeDtypeStruct((N, D), jnp.float32),
    )(x, p0, p1, wn1a, wn1b, bn1, wn2, bn2, gamma, beta)


def kernel(x, edge_index, edge_attr, W1, b1, W2, b2, W3, b3,
           Wn1, bn1, Wn2, bn2, gamma, beta):
    row = edge_index[0].astype(jnp.int32).reshape(NW, NCHUNK, C)
    col_s = edge_index[1].astype(jnp.int32).reshape(NW, NCHUNK, C)

    sc_gather, sc_scatter = _build_sc_kernels()
    xr, xc = sc_gather(row, col_s, x)

    bf = jnp.bfloat16
    edge_out = _edge_mlp(
        xr, xc, edge_attr,
        W1[:D].astype(bf), W1[D:2 * D].astype(bf), W1[2 * D:].astype(bf),
        b1.reshape(1, H),
        W2.astype(bf), b2.reshape(1, H), W3.astype(bf), b3.reshape(1, D))

    partials = sc_scatter(col_s, edge_out, jnp.zeros((C, D), jnp.float32))

    return _node_mlp(
        x, partials[0], partials[1],
        Wn1[:D], Wn1[D:], bn1.reshape(1, H),
        Wn2, bn2.reshape(1, D),
        gamma.reshape(1, D), beta.reshape(1, D))


# speedup vs baseline: 1.1621x; 1.1107x over previous
import functools

import jax
import jax.numpy as jnp
from jax import lax
from jax.experimental import pallas as pl
from jax.experimental.pallas import tpu as pltpu
from jax.experimental.pallas import tpu_sc as plsc

N = 10000
E = 320000
D = 128
ED = 16
H = 128

NC = 2
NS = 16
NW = NC * NS
EW = E // NW
C = 80
NCHUNK = EW // C
N_PAD = 10240
NROWS = N_PAD // NS
NB = 4
NB_S = 3


@functools.cache
def _build_sc_kernels():
    mesh = plsc.VectorSubcoreMesh(
        core_axis_name="c", subcore_axis_name="s",
        num_cores=NC, num_subcores=NS)

    @functools.partial(
        pl.kernel,
        out_type=(jax.ShapeDtypeStruct((E, D), jnp.float32),
                  jax.ShapeDtypeStruct((E, D), jnp.float32)),
        mesh=mesh,
        scratch_types=[
            pltpu.VMEM((NCHUNK, C), jnp.int32),
            pltpu.VMEM((NCHUNK, C), jnp.int32),
            pltpu.VMEM((NB, C, D), jnp.float32),
            pltpu.VMEM((NB, C, D), jnp.float32),
        ] + [pltpu.SemaphoreType.DMA] * (2 * NB),
    )
    def sc_gather(row_hbm, col_hbm, x_hbm, xr_hbm, xc_hbm,
                  ridx_v, cidx_v, buf_r, buf_c, *sems):
        wid = lax.axis_index("s") * NC + lax.axis_index("c")
        base = wid * EW
        pltpu.sync_copy(row_hbm.at[wid], ridx_v)
        pltpu.sync_copy(col_hbm.at[wid], cidx_v)

        gsem = sems[:NB]
        ssem = sems[NB:]

        def start_gather(ch, b):
            pltpu.async_copy(x_hbm.at[ridx_v.at[ch]], buf_r.at[b], gsem[b])
            pltpu.async_copy(x_hbm.at[cidx_v.at[ch]], buf_c.at[b], gsem[b])

        def wait_gather(b):
            pltpu.make_async_copy(x_hbm.at[ridx_v.at[0]], buf_r.at[b],
                                  gsem[b]).wait()
            pltpu.make_async_copy(x_hbm.at[cidx_v.at[0]], buf_c.at[b],
                                  gsem[b]).wait()

        def start_store(ch, b):
            pltpu.async_copy(buf_r.at[b], xr_hbm.at[pl.ds(base + ch * C, C)],
                             ssem[b])
            pltpu.async_copy(buf_c.at[b], xc_hbm.at[pl.ds(base + ch * C, C)],
                             ssem[b])

        def wait_store(b):
            pltpu.make_async_copy(buf_r.at[b], xr_hbm.at[pl.ds(base, C)],
                                  ssem[b]).wait()
            pltpu.make_async_copy(buf_c.at[b], xc_hbm.at[pl.ds(base, C)],
                                  ssem[b]).wait()

        for b in range(NB):
            start_gather(b, b)

        def body(i, _):
            ch = NB * i
            for k in range(NB):
                wait_gather(k)
                start_store(ch + k, k)
            for k in range(NB):
                wait_store(k)

                @pl.when(ch + NB + k < NCHUNK)
                def _(k=k):
                    start_gather(ch + NB + k, k)

            return 0

        lax.fori_loop(0, NCHUNK // NB, body, 0)
        wait_gather(0)
        start_store(NCHUNK - 1, 0)
        wait_store(0)

    @functools.partial(
        pl.kernel,
        out_type=jax.ShapeDtypeStruct((NC, N_PAD, D), jnp.float32),
        mesh=mesh,
        scratch_types=[
            pltpu.VMEM((NCHUNK, C), jnp.int32),
            pltpu.VMEM((NB_S, C, D), jnp.float32),
            pltpu.VMEM_SHARED((N_PAD, D), jnp.float32),
        ] + [pltpu.SemaphoreType.DMA] * (2 * NB_S),
    )
    def sc_scatter(col_hbm, eo_hbm, zeros_hbm, out_hbm,
                   cidx_v, buf, aggr_sh, *sems):
        c = lax.axis_index("c")
        s = lax.axis_index("s")
        wid = s * NC + c
        base = wid * EW
        rbase = s * NROWS

        pltpu.sync_copy(zeros_hbm, buf.at[0])

        def zbody(k, _):
            pltpu.sync_copy(buf.at[0], aggr_sh.at[pl.ds(rbase + k * C, C)])
            return 0

        lax.fori_loop(0, NROWS // C, zbody, 0)
        plsc.subcore_barrier()

        pltpu.sync_copy(col_hbm.at[wid], cidx_v)

        lsem = sems[:NB_S]
        asem = sems[NB_S:]

        def start_load(ch, b):
            pltpu.async_copy(eo_hbm.at[pl.ds(base + ch * C, C)], buf.at[b],
                             lsem[b])

        def wait_load(b):
            pltpu.make_async_copy(eo_hbm.at[pl.ds(base, C)], buf.at[b],
                                  lsem[b]).wait()

        def do_add(ch, b):
            pltpu.async_copy(buf.at[b], aggr_sh.at[cidx_v.at[ch]], asem[b],
                             add=True)
            pltpu.make_async_copy(buf.at[b], aggr_sh.at[pl.ds(0, C)],
                                  asem[b]).wait()

        for b in range(NB_S):
            start_load(b, b)

        def body(i, _):
            ch = NB_S * i
            for k in range(NB_S):
                wait_load(k)
                do_add(ch + k, k)

                @pl.when(ch + NB_S + k < NCHUNK)
                def _(k=k):
                    start_load(ch + NB_S + k, k)

            return 0

        lax.fori_loop(0, NCHUNK // NB_S, body, 0)
        wait_load(0)
        do_add(NCHUNK - 2, 0)
        wait_load(1)
        do_add(NCHUNK - 1, 1)
        plsc.subcore_barrier()

        def wbody(k, _):
            pltpu.sync_copy(aggr_sh.at[pl.ds(rbase + k * C, C)], buf.at[0])
            pltpu.sync_copy(buf.at[0],
                            out_hbm.at[c].at[pl.ds(rbase + k * C, C)])
            return 0

        lax.fori_loop(0, NROWS // C, wbody, 0)

    return sc_gather, sc_scatter


BE = 4000


def _edge_mlp_body(xr, xc, ea, w1r, w1c, w1e, b1, w2, b2, w3, b3, out):
    bf = jnp.bfloat16
    h = (jnp.dot(xr[...].astype(bf), w1r[...],
                 preferred_element_type=jnp.float32)
         + jnp.dot(xc[...].astype(bf), w1c[...],
                   preferred_element_type=jnp.float32)
         + jnp.dot(ea[...].astype(bf), w1e[...],
                   preferred_element_type=jnp.float32)
         + b1[...])
    h = jnp.maximum(h, 0.0).astype(bf)
    h = jnp.maximum(
        jnp.dot(h, w2[...], preferred_element_type=jnp.float32) + b2[...],
        0.0).astype(bf)
    out[...] = jnp.dot(h, w3[...], preferred_element_type=jnp.float32) + b3[...]


def _edge_mlp(xr, xc, ea, w1r, w1c, w1e, b1, w2, b2, w3, b3):
    full = lambda shape: pl.BlockSpec(shape, lambda i: (0, 0))
    return pl.pallas_call(
        _edge_mlp_body,
        grid=(E // BE,),
        in_specs=[
            pl.BlockSpec((BE, D), lambda i: (i, 0)),
            pl.BlockSpec((BE, D), lambda i: (i, 0)),
            pl.BlockSpec((BE, ED), lambda i: (i, 0)),
            full((D, H)), full((D, H)), full((ED, H)), full((1, H)),
            full((H, H)), full((1, H)),
            full((H, D)), full((1, D)),
        ],
        out_specs=pl.BlockSpec((BE, D), lambda i: (i, 0)),
        out_shape=jax.ShapeDtypeStruct((E, D), jnp.float32),
    )(xr, xc, ea, w1r, w1c, w1e, b1, w2, b2, w3, b3)


BN = 2000


def _node_body(x, p0, p1, wn1a, wn1b, bn1, wn2, bn2, gamma, beta, out):
    xb = x[...]
    aggr = p0[...] + p1[...]
    g = jnp.maximum(
        jnp.dot(xb, wn1a[...], preferred_element_type=jnp.float32)
        + jnp.dot(aggr, wn1b[...], preferred_element_type=jnp.float32)
        + bn1[...], 0.0)
    h = xb + jnp.dot(g, wn2[...], preferred_element_type=jnp.float32) + bn2[...]
    mu = jnp.mean(h, axis=-1, keepdims=True)
    var = jnp.mean((h - mu) ** 2, axis=-1, keepdims=True)
    out[...] = (h - mu) * jax.lax.rsqrt(var + 1e-5) * gamma[...] + beta[...]


def _node_mlp(x, p0, p1, wn1a, wn1b, bn1, wn2, bn2, gamma, beta):
    full = lambda shape: pl.BlockSpec(shape, lambda i: (0, 0))
    return pl.pallas_call(
        _node_body,
        grid=(N // BN,),
        in_specs=[
            pl.BlockSpec((BN, D), lambda i: (i, 0)),
            pl.BlockSpec((BN, D), lambda i: (i, 0)),
            pl.BlockSpec((BN, D), lambda i: (i, 0)),
            full((D, H)), full((D, H)), full((1, H)),
            full((H, D)), full((1, D)),
            full((1, D)), full((1, D)),
        ],
        out_specs=pl.BlockSpec((BN, D), lambda i: (i, 0)),
        out_shape=jax.ShapeDtypeStruct((N, D), jnp.float32),
    )(x, p0, p1, wn1a, wn1b, bn1, wn2, bn2, gamma, beta)


def kernel(x, edge_index, edge_attr, W1, b1, W2, b2, W3, b3,
           Wn1, bn1, Wn2, bn2, gamma, beta):
    row = edge_index[0].astype(jnp.int32).reshape(NW, NCHUNK, C)
    col_s = edge_index[1].astype(jnp.int32).reshape(NW, NCHUNK, C)

    sc_gather, sc_scatter = _build_sc_kernels()
    xr, xc = sc_gather(row, col_s, x)

    bf = jnp.bfloat16
    edge_out = _edge_mlp(
        xr, xc, edge_attr,
        W1[:D].astype(bf), W1[D:2 * D].astype(bf), W1[2 * D:].astype(bf),
        b1.reshape(1, H),
        W2.astype(bf), b2.reshape(1, H), W3.astype(bf), b3.reshape(1, D))

    partials = sc_scatter(col_s, edge_out, jnp.zeros((C, D), jnp.float32))

    return _node_mlp(
        x, partials[0], partials[1],
        Wn1[:D], Wn1[D:], bn1.reshape(1, H),
        Wn2, bn2.reshape(1, D),
        gamma.reshape(1, D), beta.reshape(1, D))


# speedup vs baseline: 1.2093x; 1.0406x over previous
import functools

import jax
import jax.numpy as jnp
from jax import lax
from jax.experimental import pallas as pl
from jax.experimental.pallas import tpu as pltpu
from jax.experimental.pallas import tpu_sc as plsc

N = 10000
E = 320000
D = 128
ED = 16
H = 128

NC = 2
NS = 16
NW = NC * NS
EW = E // NW
C = 80
NCHUNK = EW // C
N_PAD = 10240
NROWS = N_PAD // NS
NB = 4
NB_S = 3


@functools.cache
def _build_sc_kernels():
    mesh = plsc.VectorSubcoreMesh(
        core_axis_name="c", subcore_axis_name="s",
        num_cores=NC, num_subcores=NS)

    @functools.partial(
        pl.kernel,
        out_type=(jax.ShapeDtypeStruct((E, D), jnp.float32),
                  jax.ShapeDtypeStruct((E, D), jnp.float32)),
        mesh=mesh,
        scratch_types=[
            pltpu.VMEM((NCHUNK, C), jnp.int32),
            pltpu.VMEM((NCHUNK, C), jnp.int32),
            pltpu.VMEM((NB, C, D), jnp.float32),
            pltpu.VMEM((NB, C, D), jnp.float32),
        ] + [pltpu.SemaphoreType.DMA] * (2 * NB),
    )
    def sc_gather(row_hbm, col_hbm, x_hbm, xr_hbm, xc_hbm,
                  ridx_v, cidx_v, buf_r, buf_c, *sems):
        wid = lax.axis_index("s") * NC + lax.axis_index("c")
        base = wid * EW
        pltpu.sync_copy(row_hbm.at[wid], ridx_v)
        pltpu.sync_copy(col_hbm.at[wid], cidx_v)

        gsem = sems[:NB]
        ssem = sems[NB:]

        def start_gather(ch, b):
            pltpu.async_copy(x_hbm.at[ridx_v.at[ch]], buf_r.at[b], gsem[b])
            pltpu.async_copy(x_hbm.at[cidx_v.at[ch]], buf_c.at[b], gsem[b])

        def wait_gather(b):
            pltpu.make_async_copy(x_hbm.at[ridx_v.at[0]], buf_r.at[b],
                                  gsem[b]).wait()
            pltpu.make_async_copy(x_hbm.at[cidx_v.at[0]], buf_c.at[b],
                                  gsem[b]).wait()

        def start_store(ch, b):
            pltpu.async_copy(buf_r.at[b], xr_hbm.at[pl.ds(base + ch * C, C)],
                             ssem[b])
            pltpu.async_copy(buf_c.at[b], xc_hbm.at[pl.ds(base + ch * C, C)],
                             ssem[b])

        def wait_store(b):
            pltpu.make_async_copy(buf_r.at[b], xr_hbm.at[pl.ds(base, C)],
                                  ssem[b]).wait()
            pltpu.make_async_copy(buf_c.at[b], xc_hbm.at[pl.ds(base, C)],
                                  ssem[b]).wait()

        for b in range(NB):
            start_gather(b, b)

        def body(i, _):
            ch = NB * i
            for k in range(NB):
                wait_gather(k)
                start_store(ch + k, k)
            for k in range(NB):
                wait_store(k)

                @pl.when(ch + NB + k < NCHUNK)
                def _(k=k):
                    start_gather(ch + NB + k, k)

            return 0

        lax.fori_loop(0, NCHUNK // NB, body, 0)
        wait_gather(0)
        start_store(NCHUNK - 1, 0)
        wait_store(0)

    @functools.partial(
        pl.kernel,
        out_type=jax.ShapeDtypeStruct((NC, N_PAD, D), jnp.float32),
        mesh=mesh,
        scratch_types=[
            pltpu.VMEM((NCHUNK, C), jnp.int32),
            pltpu.VMEM((NB_S, C, D), jnp.float32),
            pltpu.VMEM_SHARED((N_PAD, D), jnp.float32),
        ] + [pltpu.SemaphoreType.DMA] * (2 * NB_S),
    )
    def sc_scatter(col_hbm, eo_hbm, zeros_hbm, out_hbm,
                   cidx_v, buf, aggr_sh, *sems):
        c = lax.axis_index("c")
        s = lax.axis_index("s")
        wid = s * NC + c
        base = wid * EW
        rbase = s * NROWS

        pltpu.sync_copy(zeros_hbm, buf.at[0])

        def zbody(k, _):
            pltpu.sync_copy(buf.at[0], aggr_sh.at[pl.ds(rbase + k * C, C)])
            return 0

        lax.fori_loop(0, NROWS // C, zbody, 0)
        plsc.subcore_barrier()

        pltpu.sync_copy(col_hbm.at[wid], cidx_v)

        lsem = sems[:NB_S]
        asem = sems[NB_S:]

        def start_load(ch, b):
            pltpu.async_copy(eo_hbm.at[pl.ds(base + ch * C, C)], buf.at[b],
                             lsem[b])

        def wait_load(b):
            pltpu.make_async_copy(eo_hbm.at[pl.ds(base, C)], buf.at[b],
                                  lsem[b]).wait()

        def do_add(ch, b):
            pltpu.async_copy(buf.at[b], aggr_sh.at[cidx_v.at[ch]], asem[b],
                             add=True)
            pltpu.make_async_copy(buf.at[b], aggr_sh.at[pl.ds(0, C)],
                                  asem[b]).wait()

        for b in range(NB_S):
            start_load(b, b)

        def body(i, _):
            ch = NB_S * i
            for k in range(NB_S):
                wait_load(k)
                do_add(ch + k, k)

                @pl.when(ch + NB_S + k < NCHUNK)
                def _(k=k):
                    start_load(ch + NB_S + k, k)

            return 0

        lax.fori_loop(0, NCHUNK // NB_S, body, 0)
        wait_load(0)
        do_add(NCHUNK - 2, 0)
        wait_load(1)
        do_add(NCHUNK - 1, 1)
        plsc.subcore_barrier()

        def wbody(k, _):
            pltpu.sync_copy(aggr_sh.at[pl.ds(rbase + k * C, C)], buf.at[0])
            pltpu.sync_copy(buf.at[0],
                            out_hbm.at[c].at[pl.ds(rbase + k * C, C)])
            return 0

        lax.fori_loop(0, NROWS // C, wbody, 0)

    return sc_gather, sc_scatter


BE = 8000


def _edge_mlp_body(xr, xc, ea, w1r, w1c, w1e, b1, w2, b2, w3, b3, out):
    bf = jnp.bfloat16
    h = (jnp.dot(xr[...].astype(bf), w1r[...],
                 preferred_element_type=jnp.float32)
         + jnp.dot(xc[...].astype(bf), w1c[...],
                   preferred_element_type=jnp.float32)
         + jnp.dot(ea[...].astype(bf), w1e[...],
                   preferred_element_type=jnp.float32)
         + b1[...])
    h = jnp.maximum(h, 0.0).astype(bf)
    h = jnp.maximum(
        jnp.dot(h, w2[...], preferred_element_type=jnp.float32) + b2[...],
        0.0).astype(bf)
    out[...] = jnp.dot(h, w3[...], preferred_element_type=jnp.float32) + b3[...]


def _edge_mlp(xr, xc, ea, w1r, w1c, w1e, b1, w2, b2, w3, b3):
    full = lambda shape: pl.BlockSpec(shape, lambda i: (0, 0))
    return pl.pallas_call(
        _edge_mlp_body,
        grid=(E // BE,),
        in_specs=[
            pl.BlockSpec((BE, D), lambda i: (i, 0)),
            pl.BlockSpec((BE, D), lambda i: (i, 0)),
            pl.BlockSpec((BE, ED), lambda i: (i, 0)),
            full((D, H)), full((D, H)), full((ED, H)), full((1, H)),
            full((H, H)), full((1, H)),
            full((H, D)), full((1, D)),
        ],
        out_specs=pl.BlockSpec((BE, D), lambda i: (i, 0)),
        out_shape=jax.ShapeDtypeStruct((E, D), jnp.float32),
    )(xr, xc, ea, w1r, w1c, w1e, b1, w2, b2, w3, b3)


BN = 2000


def _node_body(x, p0, p1, wn1a, wn1b, bn1, wn2, bn2, gamma, beta, out):
    xb = x[...]
    aggr = p0[...] + p1[...]
    g = jnp.maximum(
        jnp.dot(xb, wn1a[...], preferred_element_type=jnp.float32)
        + jnp.dot(aggr, wn1b[...], preferred_element_type=jnp.float32)
        + bn1[...], 0.0)
    h = xb + jnp.dot(g, wn2[...], preferred_element_type=jnp.float32) + bn2[...]
    mu = jnp.mean(h, axis=-1, keepdims=True)
    var = jnp.mean((h - mu) ** 2, axis=-1, keepdims=True)
    out[...] = (h - mu) * jax.lax.rsqrt(var + 1e-5) * gamma[...] + beta[...]


def _node_mlp(x, p0, p1, wn1a, wn1b, bn1, wn2, bn2, gamma, beta):
    full = lambda shape: pl.BlockSpec(shape, lambda i: (0, 0))
    return pl.pallas_call(
        _node_body,
        grid=(N // BN,),
        in_specs=[
            pl.BlockSpec((BN, D), lambda i: (i, 0)),
            pl.BlockSpec((BN, D), lambda i: (i, 0)),
            pl.BlockSpec((BN, D), lambda i: (i, 0)),
            full((D, H)), full((D, H)), full((1, H)),
            full((H, D)), full((1, D)),
            full((1, D)), full((1, D)),
        ],
        out_specs=pl.BlockSpec((BN, D), lambda i: (i, 0)),
        out_shape=jax.ShapeDtypeStruct((N, D), jnp.float32),
    )(x, p0, p1, wn1a, wn1b, bn1, wn2, bn2, gamma, beta)


def kernel(x, edge_index, edge_attr, W1, b1, W2, b2, W3, b3,
           Wn1, bn1, Wn2, bn2, gamma, beta):
    row = edge_index[0].astype(jnp.int32).reshape(NW, NCHUNK, C)
    col_s = edge_index[1].astype(jnp.int32).reshape(NW, NCHUNK, C)

    sc_gather, sc_scatter = _build_sc_kernels()
    xr, xc = sc_gather(row, col_s, x)

    bf = jnp.bfloat16
    edge_out = _edge_mlp(
        xr, xc, edge_attr,
        W1[:D].astype(bf), W1[D:2 * D].astype(bf), W1[2 * D:].astype(bf),
        b1.reshape(1, H),
        W2.astype(bf), b2.reshape(1, H), W3.astype(bf), b3.reshape(1, D))

    partials = sc_scatter(col_s, edge_out, jnp.zeros((C, D), jnp.float32))

    return _node_mlp(
        x, partials[0], partials[1],
        Wn1[:D], Wn1[D:], bn1.reshape(1, H),
        Wn2, bn2.reshape(1, D),
        gamma.reshape(1, D), beta.reshape(1, D))
